# baseline (speedup 1.0000x reference)
import functools

import jax
import jax.numpy as jnp
from jax import lax
from jax.experimental import pallas as pl
from jax.experimental.pallas import tpu as pltpu
from jax.experimental.pallas import tpu_sc as plsc

N = 10000
E = 320000
D = 128
CHUNK = 50
NC, NS = 2, 16
ROWS = E // CHUNK
RPT = ROWS // NS
IB = 40
NBLK = RPT // IB
NPAD = 10240
HSTRIPE = NPAD // NS
VSTRIPE = 624
VTAIL = N - NS * VSTRIPE

_MESH = plsc.VectorSubcoreMesh(
    core_axis_name="c", subcore_axis_name="s", num_cores=NC, num_subcores=NS
)


@functools.partial(
    pl.kernel,
    out_type=(
        jax.ShapeDtypeStruct((NPAD,), jnp.float32),
        jax.ShapeDtypeStruct((NPAD,), jnp.float32),
    ),
    mesh=_MESH,
    scratch_types=[
        pltpu.VMEM((RPT, CHUNK), jnp.int32),
        pltpu.VMEM((112,), jnp.float32),
        pltpu.VMEM((HSTRIPE,), jnp.float32),
        pltpu.VMEM_SHARED((NPAD,), jnp.float32),
    ],
)
def _sc_degrees(row2d, col2d, out_row, out_col, idx_v, ones_v, zbuf, hist):
    c = lax.axis_index("c")
    sid = lax.axis_index("s")

    for j in range(7):
        ones_v[pl.ds(j * 16, 16)] = jnp.ones((16,), jnp.float32)

    def _z(i, carry):
        zbuf[pl.ds(i * 16, 16)] = jnp.zeros((16,), jnp.float32)
        return carry

    lax.fori_loop(0, HSTRIPE // 16, _z, 0)
    pltpu.sync_copy(zbuf, hist.at[pl.ds(sid * HSTRIPE, HSTRIPE)])
    plsc.subcore_barrier()

    def _accum(src2d):
        pltpu.sync_copy(src2d.at[pl.ds(sid * RPT, RPT)], idx_v)

        def _step(i, carry):
            pltpu.sync_copy(
                ones_v.at[pl.ds(0, CHUNK)], hist.at[idx_v.at[i]], add=True
            )
            return carry

        lax.fori_loop(0, RPT, _step, 0)

    pl.when(c == 0)(lambda: _accum(row2d))
    pl.when(c == 1)(lambda: _accum(col2d))
    plsc.subcore_barrier()

    sl = pl.ds(sid * HSTRIPE, HSTRIPE)
    pl.when(c == 0)(lambda: pltpu.sync_copy(hist.at[sl], out_row.at[sl]))
    pl.when(c == 1)(lambda: pltpu.sync_copy(hist.at[sl], out_col.at[sl]))


@functools.partial(
    pl.kernel,
    out_type=(
        jax.ShapeDtypeStruct((N, D), jnp.float32),
        jax.ShapeDtypeStruct((N, D), jnp.float32),
    ),
    mesh=_MESH,
    scratch_types=[
        pltpu.VMEM((IB, CHUNK), jnp.int32),
        pltpu.VMEM((IB, CHUNK), jnp.int32),
        pltpu.VMEM((2, CHUNK, D), jnp.float32),
        pltpu.VMEM_SHARED((N, D), jnp.float32),
        pltpu.SemaphoreType.DMA,
        pltpu.SemaphoreType.DMA,
    ],
)
def _sc_spmm(x_t, x_s, row2d, col2d, out_s, out_t,
             gidx, sidx, buf, acc, gsem, ssem):
    c = lax.axis_index("c")
    sid = lax.axis_index("s")

    def _striped_copy(src, dst):
        rsl = pl.ds(sid * VSTRIPE, VSTRIPE)
        pltpu.sync_copy(src.at[rsl], dst.at[rsl])

        @pl.when(sid == NS - 1)
        def _tail():
            tsl = pl.ds(NS * VSTRIPE, VTAIL)
            pltpu.sync_copy(src.at[tsl], dst.at[tsl])

    def _run(src_hbm, gather2d, scatter2d, out_hbm):
        _striped_copy(src_hbm, acc)
        plsc.subcore_barrier()

        for b in range(NBLK):
            bsl = pl.ds(sid * RPT + b * IB, IB)
            pltpu.sync_copy(gather2d.at[bsl], gidx)
            pltpu.sync_copy(scatter2d.at[bsl], sidx)

            pltpu.async_copy(src_hbm.at[gidx.at[0]], buf.at[0], gsem)

            def _step(i, carry):
                pltpu.make_async_copy(
                    src_hbm.at[gidx.at[i]], buf.at[i % 2], gsem
                ).wait()
                pltpu.async_copy(
                    buf.at[i % 2], acc.at[sidx.at[i]], ssem, add=True
                )

                @pl.when(i + 1 < IB)
                def _prefetch():
                    @pl.when(i > 0)
                    def _drain_prev():
                        pltpu.make_async_copy(
                            buf.at[(i + 1) % 2], acc.at[sidx.at[i]], ssem
                        ).wait()

                    pltpu.async_copy(
                        src_hbm.at[gidx.at[i + 1]], buf.at[(i + 1) % 2], gsem
                    )

                return carry

            lax.fori_loop(0, IB, _step, 0)
            pltpu.make_async_copy(buf.at[0], acc.at[sidx.at[0]], ssem).wait()
            pltpu.make_async_copy(buf.at[1], acc.at[sidx.at[0]], ssem).wait()
        plsc.subcore_barrier()
        _striped_copy(acc, out_hbm)

    pl.when(c == 0)(lambda: _run(x_t, col2d, row2d, out_s))
    pl.when(c == 1)(lambda: _run(x_s, row2d, col2d, out_t))


BM = 400
GRID = N // BM

_row_spec = pl.BlockSpec((BM, D), lambda i: (i, 0))
_w_spec = pl.BlockSpec((D, D), lambda i: (0, 0))
_b_spec = pl.BlockSpec((1, D), lambda i: (0, 0))
_col_spec = pl.BlockSpec((BM, 1), lambda i: (i, 0))
_s2_spec = pl.BlockSpec((1, 2), lambda i: (0, 0))


def _mlp_scale_body(s_ref, t_ref, ws1, bs1, ws2, bs2, wt1, bt1, wt2, bt2,
                    hr_ref, hc_ref,
                    s0_ref, t0_ref, s0s_ref, t0s_ref, oi_ref, ii_ref):
    x = s_ref[...]
    h = jnp.maximum(jnp.dot(x, ws1[...], preferred_element_type=jnp.float32)
                    + bs1[...], 0.0)
    s0 = jnp.dot(h, ws2[...], preferred_element_type=jnp.float32) + bs2[...]
    y = t_ref[...]
    g = jnp.maximum(jnp.dot(y, wt1[...], preferred_element_type=jnp.float32)
                    + bt1[...], 0.0)
    t0 = jnp.dot(g, wt2[...], preferred_element_type=jnp.float32) + bt2[...]
    oi = lax.rsqrt(hr_ref[...] + 1.0)
    ii = lax.rsqrt(hc_ref[...] + 1.0)
    s0_ref[...] = s0
    t0_ref[...] = t0
    s0s_ref[...] = s0 * oi
    t0s_ref[...] = t0 * ii
    oi_ref[...] = oi
    ii_ref[...] = ii


_mlp_scale = pl.pallas_call(
    _mlp_scale_body,
    grid=(GRID,),
    in_specs=[_row_spec, _row_spec,
              _w_spec, _b_spec, _w_spec, _b_spec,
              _w_spec, _b_spec, _w_spec, _b_spec,
              _col_spec, _col_spec],
    out_specs=[_row_spec, _row_spec, _row_spec, _row_spec,
               _col_spec, _col_spec],
    out_shape=[jax.ShapeDtypeStruct((N, D), jnp.float32)] * 4
    + [jax.ShapeDtypeStruct((N, 1), jnp.float32)] * 2,
)


def _combine_body(s_ref, t_ref, accs_ref, acct_ref, oi_ref, ii_ref, w_ref,
                  s1_ref, t1_ref, s1s_ref, t1s_ref):
    oi = oi_ref[...]
    ii = ii_ref[...]
    s1 = s_ref[...] + w_ref[0, 0] * (oi * accs_ref[...])
    t1 = t_ref[...] + w_ref[0, 1] * (ii * acct_ref[...])
    s1_ref[...] = s1
    t1_ref[...] = t1
    s1s_ref[...] = s1 * oi
    t1s_ref[...] = t1 * ii


_combine_scale = pl.pallas_call(
    _combine_body,
    grid=(GRID,),
    in_specs=[_row_spec, _row_spec, _row_spec, _row_spec,
              _col_spec, _col_spec, _s2_spec],
    out_specs=[_row_spec] * 4,
    out_shape=[jax.ShapeDtypeStruct((N, D), jnp.float32)] * 4,
)


def _final_body(s_ref, t_ref, accs_ref, acct_ref, oi_ref, ii_ref, w_ref,
                s2_ref, t2_ref):
    s2_ref[...] = s_ref[...] + w_ref[0, 0] * (oi_ref[...] * accs_ref[...])
    t2_ref[...] = t_ref[...] + w_ref[0, 1] * (ii_ref[...] * acct_ref[...])


_final_combine = pl.pallas_call(
    _final_body,
    grid=(GRID,),
    in_specs=[_row_spec, _row_spec, _row_spec, _row_spec,
              _col_spec, _col_spec, _s2_spec],
    out_specs=[_row_spec] * 2,
    out_shape=[jax.ShapeDtypeStruct((N, D), jnp.float32)] * 2,
)


def kernel(s, t, edge_index, Ws1, bs1, Ws2, bs2, Wt1, bt1, Wt2, bt2, w1, w2):
    row2d = edge_index[0].reshape(ROWS, CHUNK)
    col2d = edge_index[1].reshape(ROWS, CHUNK)

    hr, hc = _sc_degrees(row2d, col2d)
    hr = hr[:N].reshape(N, 1)
    hc = hc[:N].reshape(N, 1)

    s0, t0, s0s, t0s, oi, ii = _mlp_scale(
        s, t,
        Ws1.T, bs1.reshape(1, D), Ws2.T, bs2.reshape(1, D),
        Wt1.T, bt1.reshape(1, D), Wt2.T, bt2.reshape(1, D),
        hr, hc,
    )

    acc_s, acc_t = _sc_spmm(t0s, s0s, row2d, col2d)
    s1, t1, s1s, t1s = _combine_scale(s0, t0, acc_s, acc_t, oi, ii,
                                      w1.reshape(1, 2))
    acc_s2, acc_t2 = _sc_spmm(t1s, s1s, row2d, col2d)
    s2, t2 = _final_combine(s1, t1, acc_s2, acc_t2, oi, ii, w2.reshape(1, 2))
    return (s2, t2)


# speedup vs baseline: 1.5180x; 1.5180x over previous
import functools

import jax
import jax.numpy as jnp
from jax import lax
from jax.experimental import pallas as pl
from jax.experimental.pallas import tpu as pltpu
from jax.experimental.pallas import tpu_sc as plsc

N = 10000
E = 320000
D = 128
CHUNK = 125
NC, NS = 2, 16
ROWS = E // CHUNK
RPT = ROWS // NS
IB = 40
NBLK = RPT // IB
NPAD = 10240
HSTRIPE = NPAD // NS
VSTRIPE = 624
VTAIL = N - NS * VSTRIPE

_MESH = plsc.VectorSubcoreMesh(
    core_axis_name="c", subcore_axis_name="s", num_cores=NC, num_subcores=NS
)


@functools.partial(
    pl.kernel,
    out_type=(
        jax.ShapeDtypeStruct((NPAD,), jnp.float32),
        jax.ShapeDtypeStruct((NPAD,), jnp.float32),
    ),
    mesh=_MESH,
    scratch_types=[
        pltpu.VMEM((RPT, CHUNK), jnp.int32),
        pltpu.VMEM((112,), jnp.float32),
        pltpu.VMEM((HSTRIPE,), jnp.float32),
        pltpu.VMEM_SHARED((NPAD,), jnp.float32),
    ],
)
def _sc_degrees(row2d, col2d, out_row, out_col, idx_v, ones_v, zbuf, hist):
    c = lax.axis_index("c")
    sid = lax.axis_index("s")

    for j in range(7):
        ones_v[pl.ds(j * 16, 16)] = jnp.ones((16,), jnp.float32)

    def _z(i, carry):
        zbuf[pl.ds(i * 16, 16)] = jnp.zeros((16,), jnp.float32)
        return carry

    lax.fori_loop(0, HSTRIPE // 16, _z, 0)
    pltpu.sync_copy(zbuf, hist.at[pl.ds(sid * HSTRIPE, HSTRIPE)])
    plsc.subcore_barrier()

    def _accum(src2d):
        pltpu.sync_copy(src2d.at[pl.ds(sid * RPT, RPT)], idx_v)

        def _step(i, carry):
            pltpu.sync_copy(
                ones_v.at[pl.ds(0, CHUNK)], hist.at[idx_v.at[i]], add=True
            )
            return carry

        lax.fori_loop(0, RPT, _step, 0)

    pl.when(c == 0)(lambda: _accum(row2d))
    pl.when(c == 1)(lambda: _accum(col2d))
    plsc.subcore_barrier()

    sl = pl.ds(sid * HSTRIPE, HSTRIPE)
    pl.when(c == 0)(lambda: pltpu.sync_copy(hist.at[sl], out_row.at[sl]))
    pl.when(c == 1)(lambda: pltpu.sync_copy(hist.at[sl], out_col.at[sl]))


@functools.partial(
    pl.kernel,
    out_type=(
        jax.ShapeDtypeStruct((N, D), jnp.float32),
        jax.ShapeDtypeStruct((N, D), jnp.float32),
    ),
    mesh=_MESH,
    scratch_types=[
        pltpu.VMEM((IB, CHUNK), jnp.int32),
        pltpu.VMEM((IB, CHUNK), jnp.int32),
        pltpu.VMEM((2, CHUNK, D), jnp.float32),
        pltpu.VMEM_SHARED((N, D), jnp.float32),
        pltpu.SemaphoreType.DMA,
        pltpu.SemaphoreType.DMA,
    ],
)
def _sc_spmm(x_t, x_s, row2d, col2d, out_s, out_t,
             gidx, sidx, buf, acc, gsem, ssem):
    c = lax.axis_index("c")
    sid = lax.axis_index("s")

    def _striped_copy(src, dst):
        rsl = pl.ds(sid * VSTRIPE, VSTRIPE)
        pltpu.sync_copy(src.at[rsl], dst.at[rsl])

        @pl.when(sid == NS - 1)
        def _tail():
            tsl = pl.ds(NS * VSTRIPE, VTAIL)
            pltpu.sync_copy(src.at[tsl], dst.at[tsl])

    def _run(src_hbm, gather2d, scatter2d, out_hbm):
        _striped_copy(src_hbm, acc)
        plsc.subcore_barrier()

        for b in range(NBLK):
            bsl = pl.ds(sid * RPT + b * IB, IB)
            pltpu.sync_copy(gather2d.at[bsl], gidx)
            pltpu.sync_copy(scatter2d.at[bsl], sidx)

            pltpu.async_copy(src_hbm.at[gidx.at[0]], buf.at[0], gsem)

            def _step(i, carry):
                pltpu.make_async_copy(
                    src_hbm.at[gidx.at[i]], buf.at[i % 2], gsem
                ).wait()
                pltpu.async_copy(
                    buf.at[i % 2], acc.at[sidx.at[i]], ssem, add=True
                )

                @pl.when(i + 1 < IB)
                def _prefetch():
                    @pl.when(i > 0)
                    def _drain_prev():
                        pltpu.make_async_copy(
                            buf.at[(i + 1) % 2], acc.at[sidx.at[i]], ssem
                        ).wait()

                    pltpu.async_copy(
                        src_hbm.at[gidx.at[i + 1]], buf.at[(i + 1) % 2], gsem
                    )

                return carry

            lax.fori_loop(0, IB, _step, 0)
            pltpu.make_async_copy(buf.at[0], acc.at[sidx.at[0]], ssem).wait()
            pltpu.make_async_copy(buf.at[1], acc.at[sidx.at[0]], ssem).wait()
        plsc.subcore_barrier()
        _striped_copy(acc, out_hbm)

    pl.when(c == 0)(lambda: _run(x_t, col2d, row2d, out_s))
    pl.when(c == 1)(lambda: _run(x_s, row2d, col2d, out_t))


BM = 400
GRID = N // BM

_row_spec = pl.BlockSpec((BM, D), lambda i: (i, 0))
_w_spec = pl.BlockSpec((D, D), lambda i: (0, 0))
_b_spec = pl.BlockSpec((1, D), lambda i: (0, 0))
_col_spec = pl.BlockSpec((BM, 1), lambda i: (i, 0))
_s2_spec = pl.BlockSpec((1, 2), lambda i: (0, 0))


def _mlp_scale_body(s_ref, t_ref, ws1, bs1, ws2, bs2, wt1, bt1, wt2, bt2,
                    hr_ref, hc_ref,
                    s0_ref, t0_ref, s0s_ref, t0s_ref, oi_ref, ii_ref):
    x = s_ref[...]
    h = jnp.maximum(jnp.dot(x, ws1[...], preferred_element_type=jnp.float32)
                    + bs1[...], 0.0)
    s0 = jnp.dot(h, ws2[...], preferred_element_type=jnp.float32) + bs2[...]
    y = t_ref[...]
    g = jnp.maximum(jnp.dot(y, wt1[...], preferred_element_type=jnp.float32)
                    + bt1[...], 0.0)
    t0 = jnp.dot(g, wt2[...], preferred_element_type=jnp.float32) + bt2[...]
    oi = lax.rsqrt(hr_ref[...] + 1.0)
    ii = lax.rsqrt(hc_ref[...] + 1.0)
    s0_ref[...] = s0
    t0_ref[...] = t0
    s0s_ref[...] = s0 * oi
    t0s_ref[...] = t0 * ii
    oi_ref[...] = oi
    ii_ref[...] = ii


_mlp_scale = pl.pallas_call(
    _mlp_scale_body,
    grid=(GRID,),
    in_specs=[_row_spec, _row_spec,
              _w_spec, _b_spec, _w_spec, _b_spec,
              _w_spec, _b_spec, _w_spec, _b_spec,
              _col_spec, _col_spec],
    out_specs=[_row_spec, _row_spec, _row_spec, _row_spec,
               _col_spec, _col_spec],
    out_shape=[jax.ShapeDtypeStruct((N, D), jnp.float32)] * 4
    + [jax.ShapeDtypeStruct((N, 1), jnp.float32)] * 2,
)


def _combine_body(s_ref, t_ref, accs_ref, acct_ref, oi_ref, ii_ref, w_ref,
                  s1_ref, t1_ref, s1s_ref, t1s_ref):
    oi = oi_ref[...]
    ii = ii_ref[...]
    s1 = s_ref[...] + w_ref[0, 0] * (oi * accs_ref[...])
    t1 = t_ref[...] + w_ref[0, 1] * (ii * acct_ref[...])
    s1_ref[...] = s1
    t1_ref[...] = t1
    s1s_ref[...] = s1 * oi
    t1s_ref[...] = t1 * ii


_combine_scale = pl.pallas_call(
    _combine_body,
    grid=(GRID,),
    in_specs=[_row_spec, _row_spec, _row_spec, _row_spec,
              _col_spec, _col_spec, _s2_spec],
    out_specs=[_row_spec] * 4,
    out_shape=[jax.ShapeDtypeStruct((N, D), jnp.float32)] * 4,
)


def _final_body(s_ref, t_ref, accs_ref, acct_ref, oi_ref, ii_ref, w_ref,
                s2_ref, t2_ref):
    s2_ref[...] = s_ref[...] + w_ref[0, 0] * (oi_ref[...] * accs_ref[...])
    t2_ref[...] = t_ref[...] + w_ref[0, 1] * (ii_ref[...] * acct_ref[...])


_final_combine = pl.pallas_call(
    _final_body,
    grid=(GRID,),
    in_specs=[_row_spec, _row_spec, _row_spec, _row_spec,
              _col_spec, _col_spec, _s2_spec],
    out_specs=[_row_spec] * 2,
    out_shape=[jax.ShapeDtypeStruct((N, D), jnp.float32)] * 2,
)


def kernel(s, t, edge_index, Ws1, bs1, Ws2, bs2, Wt1, bt1, Wt2, bt2, w1, w2):
    row2d = edge_index[0].reshape(ROWS, CHUNK)
    col2d = edge_index[1].reshape(ROWS, CHUNK)

    hr, hc = _sc_degrees(row2d, col2d)
    hr = hr[:N].reshape(N, 1)
    hc = hc[:N].reshape(N, 1)

    s0, t0, s0s, t0s, oi, ii = _mlp_scale(
        s, t,
        Ws1.T, bs1.reshape(1, D), Ws2.T, bs2.reshape(1, D),
        Wt1.T, bt1.reshape(1, D), Wt2.T, bt2.reshape(1, D),
        hr, hc,
    )

    acc_s, acc_t = _sc_spmm(t0s, s0s, row2d, col2d)
    s1, t1, s1s, t1s = _combine_scale(s0, t0, acc_s, acc_t, oi, ii,
                                      w1.reshape(1, 2))
    acc_s2, acc_t2 = _sc_spmm(t1s, s1s, row2d, col2d)
    s2, t2 = _final_combine(s1, t1, acc_s2, acc_t2, oi, ii, w2.reshape(1, 2))
    return (s2, t2)


# speedup vs baseline: 1.5197x; 1.0011x over previous
import functools

import jax
import jax.numpy as jnp
from jax import lax
from jax.experimental import pallas as pl
from jax.experimental.pallas import tpu as pltpu
from jax.experimental.pallas import tpu_sc as plsc

N = 10000
E = 320000
D = 128
CHUNK = 125
NC, NS = 2, 16
ROWS = E // CHUNK
RPT = ROWS // NS
IB = 40
NBLK = RPT // IB
NPAD = 10240
HSTRIPE = NPAD // NS
VSTRIPE = 624
VTAIL = N - NS * VSTRIPE

_MESH = plsc.VectorSubcoreMesh(
    core_axis_name="c", subcore_axis_name="s", num_cores=NC, num_subcores=NS
)


@functools.partial(
    pl.kernel,
    out_type=(
        jax.ShapeDtypeStruct((NPAD,), jnp.float32),
        jax.ShapeDtypeStruct((NPAD,), jnp.float32),
    ),
    mesh=_MESH,
    scratch_types=[
        pltpu.VMEM((RPT, CHUNK), jnp.int32),
        pltpu.VMEM((128,), jnp.float32),
        pltpu.VMEM((HSTRIPE,), jnp.float32),
        pltpu.VMEM_SHARED((NPAD,), jnp.float32),
    ],
)
def _sc_degrees(row2d, col2d, out_row, out_col, idx_v, ones_v, zbuf, hist):
    c = lax.axis_index("c")
    sid = lax.axis_index("s")

    for j in range(8):
        ones_v[pl.ds(j * 16, 16)] = jnp.ones((16,), jnp.float32)

    def _z(i, carry):
        zbuf[pl.ds(i * 16, 16)] = jnp.zeros((16,), jnp.float32)
        return carry

    lax.fori_loop(0, HSTRIPE // 16, _z, 0)
    pltpu.sync_copy(zbuf, hist.at[pl.ds(sid * HSTRIPE, HSTRIPE)])
    plsc.subcore_barrier()

    def _accum(src2d):
        pltpu.sync_copy(src2d.at[pl.ds(sid * RPT, RPT)], idx_v)

        def _step(i, carry):
            pltpu.sync_copy(
                ones_v.at[pl.ds(0, CHUNK)], hist.at[idx_v.at[i]], add=True
            )
            return carry

        lax.fori_loop(0, RPT, _step, 0)

    pl.when(c == 0)(lambda: _accum(row2d))
    pl.when(c == 1)(lambda: _accum(col2d))
    plsc.subcore_barrier()

    sl = pl.ds(sid * HSTRIPE, HSTRIPE)
    pl.when(c == 0)(lambda: pltpu.sync_copy(hist.at[sl], out_row.at[sl]))
    pl.when(c == 1)(lambda: pltpu.sync_copy(hist.at[sl], out_col.at[sl]))


@functools.partial(
    pl.kernel,
    out_type=(
        jax.ShapeDtypeStruct((N, D), jnp.float32),
        jax.ShapeDtypeStruct((N, D), jnp.float32),
    ),
    mesh=_MESH,
    scratch_types=[
        pltpu.VMEM((IB, CHUNK), jnp.int32),
        pltpu.VMEM((IB, CHUNK), jnp.int32),
        pltpu.VMEM((2, CHUNK, D), jnp.float32),
        pltpu.VMEM_SHARED((N, D), jnp.float32),
        pltpu.SemaphoreType.DMA,
        pltpu.SemaphoreType.DMA,
    ],
)
def _sc_spmm(x_t, x_s, row2d, col2d, out_s, out_t,
             gidx, sidx, buf, acc, gsem, ssem):
    c = lax.axis_index("c")
    sid = lax.axis_index("s")

    def _striped_copy(src, dst):
        rsl = pl.ds(sid * VSTRIPE, VSTRIPE)
        pltpu.sync_copy(src.at[rsl], dst.at[rsl])

        @pl.when(sid == NS - 1)
        def _tail():
            tsl = pl.ds(NS * VSTRIPE, VTAIL)
            pltpu.sync_copy(src.at[tsl], dst.at[tsl])

    def _run(src_hbm, gather2d, scatter2d, out_hbm):
        _striped_copy(src_hbm, acc)
        plsc.subcore_barrier()

        for b in range(NBLK):
            bsl = pl.ds(sid * RPT + b * IB, IB)
            pltpu.sync_copy(gather2d.at[bsl], gidx)
            pltpu.sync_copy(scatter2d.at[bsl], sidx)

            pltpu.async_copy(src_hbm.at[gidx.at[0]], buf.at[0], gsem)

            def _step(i, carry):
                pltpu.make_async_copy(
                    src_hbm.at[gidx.at[i]], buf.at[i % 2], gsem
                ).wait()
                pltpu.async_copy(
                    buf.at[i % 2], acc.at[sidx.at[i]], ssem, add=True
                )

                @pl.when(i + 1 < IB)
                def _prefetch():
                    @pl.when(i > 0)
                    def _drain_prev():
                        pltpu.make_async_copy(
                            buf.at[(i + 1) % 2], acc.at[sidx.at[i]], ssem
                        ).wait()

                    pltpu.async_copy(
                        src_hbm.at[gidx.at[i + 1]], buf.at[(i + 1) % 2], gsem
                    )

                return carry

            lax.fori_loop(0, IB, _step, 0)
            pltpu.make_async_copy(buf.at[0], acc.at[sidx.at[0]], ssem).wait()
            pltpu.make_async_copy(buf.at[1], acc.at[sidx.at[0]], ssem).wait()
        plsc.subcore_barrier()
        _striped_copy(acc, out_hbm)

    pl.when(c == 0)(lambda: _run(x_t, col2d, row2d, out_s))
    pl.when(c == 1)(lambda: _run(x_s, row2d, col2d, out_t))


BM = 400
GRID = N // BM

_row_spec = pl.BlockSpec((BM, D), lambda i: (i, 0))
_w_spec = pl.BlockSpec((D, D), lambda i: (0, 0))
_b_spec = pl.BlockSpec((1, D), lambda i: (0, 0))
_col_spec = pl.BlockSpec((BM, 1), lambda i: (i, 0))
_s2_spec = pl.BlockSpec((1, 2), lambda i: (0, 0))


def _mlp_scale_body(s_ref, t_ref, ws1, bs1, ws2, bs2, wt1, bt1, wt2, bt2,
                    hr_ref, hc_ref,
                    s0_ref, t0_ref, s0s_ref, t0s_ref, oi_ref, ii_ref):
    x = s_ref[...]
    h = jnp.maximum(jnp.dot(x, ws1[...], preferred_element_type=jnp.float32)
                    + bs1[...], 0.0)
    s0 = jnp.dot(h, ws2[...], preferred_element_type=jnp.float32) + bs2[...]
    y = t_ref[...]
    g = jnp.maximum(jnp.dot(y, wt1[...], preferred_element_type=jnp.float32)
                    + bt1[...], 0.0)
    t0 = jnp.dot(g, wt2[...], preferred_element_type=jnp.float32) + bt2[...]
    oi = lax.rsqrt(hr_ref[...] + 1.0)
    ii = lax.rsqrt(hc_ref[...] + 1.0)
    s0_ref[...] = s0
    t0_ref[...] = t0
    s0s_ref[...] = s0 * oi
    t0s_ref[...] = t0 * ii
    oi_ref[...] = oi
    ii_ref[...] = ii


_mlp_scale = pl.pallas_call(
    _mlp_scale_body,
    grid=(GRID,),
    in_specs=[_row_spec, _row_spec,
              _w_spec, _b_spec, _w_spec, _b_spec,
              _w_spec, _b_spec, _w_spec, _b_spec,
              _col_spec, _col_spec],
    out_specs=[_row_spec, _row_spec, _row_spec, _row_spec,
               _col_spec, _col_spec],
    out_shape=[jax.ShapeDtypeStruct((N, D), jnp.float32)] * 4
    + [jax.ShapeDtypeStruct((N, 1), jnp.float32)] * 2,
)


def _combine_body(s_ref, t_ref, accs_ref, acct_ref, oi_ref, ii_ref, w_ref,
                  s1_ref, t1_ref, s1s_ref, t1s_ref):
    oi = oi_ref[...]
    ii = ii_ref[...]
    s1 = s_ref[...] + w_ref[0, 0] * (oi * accs_ref[...])
    t1 = t_ref[...] + w_ref[0, 1] * (ii * acct_ref[...])
    s1_ref[...] = s1
    t1_ref[...] = t1
    s1s_ref[...] = s1 * oi
    t1s_ref[...] = t1 * ii


_combine_scale = pl.pallas_call(
    _combine_body,
    grid=(GRID,),
    in_specs=[_row_spec, _row_spec, _row_spec, _row_spec,
              _col_spec, _col_spec, _s2_spec],
    out_specs=[_row_spec] * 4,
    out_shape=[jax.ShapeDtypeStruct((N, D), jnp.float32)] * 4,
)


def _final_body(s_ref, t_ref, accs_ref, acct_ref, oi_ref, ii_ref, w_ref,
                s2_ref, t2_ref):
    s2_ref[...] = s_ref[...] + w_ref[0, 0] * (oi_ref[...] * accs_ref[...])
    t2_ref[...] = t_ref[...] + w_ref[0, 1] * (ii_ref[...] * acct_ref[...])


_final_combine = pl.pallas_call(
    _final_body,
    grid=(GRID,),
    in_specs=[_row_spec, _row_spec, _row_spec, _row_spec,
              _col_spec, _col_spec, _s2_spec],
    out_specs=[_row_spec] * 2,
    out_shape=[jax.ShapeDtypeStruct((N, D), jnp.float32)] * 2,
)


def kernel(s, t, edge_index, Ws1, bs1, Ws2, bs2, Wt1, bt1, Wt2, bt2, w1, w2):
    row2d = edge_index[0].reshape(ROWS, CHUNK)
    col2d = edge_index[1].reshape(ROWS, CHUNK)

    hr, hc = _sc_degrees(row2d, col2d)
    hr = hr[:N].reshape(N, 1)
    hc = hc[:N].reshape(N, 1)

    s0, t0, s0s, t0s, oi, ii = _mlp_scale(
        s, t,
        Ws1.T, bs1.reshape(1, D), Ws2.T, bs2.reshape(1, D),
        Wt1.T, bt1.reshape(1, D), Wt2.T, bt2.reshape(1, D),
        hr, hc,
    )

    acc_s, acc_t = _sc_spmm(t0s, s0s, row2d, col2d)
    s1, t1, s1s, t1s = _combine_scale(s0, t0, acc_s, acc_t, oi, ii,
                                      w1.reshape(1, 2))
    acc_s2, acc_t2 = _sc_spmm(t1s, s1s, row2d, col2d)
    s2, t2 = _final_combine(s1, t1, acc_s2, acc_t2, oi, ii, w2.reshape(1, 2))
    return (s2, t2)


# speedup vs baseline: 1.5934x; 1.0485x over previous
import functools

import jax
import jax.numpy as jnp
from jax import lax
from jax.experimental import pallas as pl
from jax.experimental.pallas import tpu as pltpu
from jax.experimental.pallas import tpu_sc as plsc

N = 10000
E = 320000
D = 128
CHUNK = 125
NC, NS = 2, 16
ROWS = E // CHUNK
RPT = ROWS // NS
IB = 40
NBLK = RPT // IB
NPAD = 10240
HSTRIPE = NPAD // NS
VSTRIPE = 624
VTAIL = N - NS * VSTRIPE

_MESH = plsc.VectorSubcoreMesh(
    core_axis_name="c", subcore_axis_name="s", num_cores=NC, num_subcores=NS
)


@functools.partial(
    pl.kernel,
    out_type=(
        jax.ShapeDtypeStruct((NPAD,), jnp.float32),
        jax.ShapeDtypeStruct((NPAD,), jnp.float32),
    ),
    mesh=_MESH,
    scratch_types=[
        pltpu.VMEM((RPT, CHUNK), jnp.int32),
        pltpu.VMEM((128,), jnp.float32),
        pltpu.VMEM((HSTRIPE,), jnp.float32),
        pltpu.VMEM_SHARED((NPAD,), jnp.float32),
    ],
)
def _sc_degrees(row2d, col2d, out_row, out_col, idx_v, ones_v, zbuf, hist):
    c = lax.axis_index("c")
    sid = lax.axis_index("s")

    for j in range(8):
        ones_v[pl.ds(j * 16, 16)] = jnp.ones((16,), jnp.float32)

    def _z(i, carry):
        zbuf[pl.ds(i * 16, 16)] = jnp.zeros((16,), jnp.float32)
        return carry

    lax.fori_loop(0, HSTRIPE // 16, _z, 0)
    pltpu.sync_copy(zbuf, hist.at[pl.ds(sid * HSTRIPE, HSTRIPE)])
    plsc.subcore_barrier()

    def _accum(src2d):
        pltpu.sync_copy(src2d.at[pl.ds(sid * RPT, RPT)], idx_v)

        def _step(i, carry):
            pltpu.sync_copy(
                ones_v.at[pl.ds(0, CHUNK)], hist.at[idx_v.at[i]], add=True
            )
            return carry

        lax.fori_loop(0, RPT, _step, 0)

    pl.when(c == 0)(lambda: _accum(row2d))
    pl.when(c == 1)(lambda: _accum(col2d))
    plsc.subcore_barrier()

    sl = pl.ds(sid * HSTRIPE, HSTRIPE)
    pl.when(c == 0)(lambda: pltpu.sync_copy(hist.at[sl], out_row.at[sl]))
    pl.when(c == 1)(lambda: pltpu.sync_copy(hist.at[sl], out_col.at[sl]))


@functools.partial(
    pl.kernel,
    out_type=(
        jax.ShapeDtypeStruct((N, D), jnp.float32),
        jax.ShapeDtypeStruct((N, D), jnp.float32),
    ),
    mesh=_MESH,
    scratch_types=[
        pltpu.VMEM((IB, CHUNK), jnp.int32),
        pltpu.VMEM((IB, CHUNK), jnp.int32),
        pltpu.VMEM((2, CHUNK, D), jnp.float32),
        pltpu.VMEM_SHARED((N, D), jnp.float32),
        pltpu.SemaphoreType.DMA,
        pltpu.SemaphoreType.DMA,
    ],
)
def _sc_spmm(x_t, x_s, row2d, col2d, out_s, out_t,
             gidx, sidx, buf, acc, gsem, ssem):
    c = lax.axis_index("c")
    sid = lax.axis_index("s")

    def _striped_copy(src, dst):
        rsl = pl.ds(sid * VSTRIPE, VSTRIPE)
        pltpu.sync_copy(src.at[rsl], dst.at[rsl])

        @pl.when(sid == NS - 1)
        def _tail():
            tsl = pl.ds(NS * VSTRIPE, VTAIL)
            pltpu.sync_copy(src.at[tsl], dst.at[tsl])

    def _run(src_hbm, gather2d, scatter2d, out_hbm):
        _striped_copy(src_hbm, acc)
        plsc.subcore_barrier()

        for b in range(NBLK):
            bsl = pl.ds(sid * RPT + b * IB, IB)
            pltpu.sync_copy(gather2d.at[bsl], gidx)
            pltpu.sync_copy(scatter2d.at[bsl], sidx)

            pltpu.async_copy(src_hbm.at[gidx.at[0]], buf.at[0], gsem)

            def _step(i, carry):
                pltpu.make_async_copy(
                    src_hbm.at[gidx.at[i]], buf.at[i % 2], gsem
                ).wait()
                pltpu.async_copy(
                    buf.at[i % 2], acc.at[sidx.at[i]], ssem, add=True
                )

                @pl.when(i + 1 < IB)
                def _prefetch():
                    @pl.when(i > 0)
                    def _drain_prev():
                        pltpu.make_async_copy(
                            buf.at[(i + 1) % 2], acc.at[sidx.at[i]], ssem
                        ).wait()

                    pltpu.async_copy(
                        src_hbm.at[gidx.at[i + 1]], buf.at[(i + 1) % 2], gsem
                    )

                return carry

            lax.fori_loop(0, IB, _step, 0)
            pltpu.make_async_copy(buf.at[0], acc.at[sidx.at[0]], ssem).wait()
            pltpu.make_async_copy(buf.at[1], acc.at[sidx.at[0]], ssem).wait()
        plsc.subcore_barrier()
        _striped_copy(acc, out_hbm)

    pl.when(c == 0)(lambda: _run(x_t, col2d, row2d, out_s))
    pl.when(c == 1)(lambda: _run(x_s, row2d, col2d, out_t))


BM = 2000
GRID = N // BM

_row_spec = pl.BlockSpec((BM, D), lambda i: (i, 0))
_w_spec = pl.BlockSpec((D, D), lambda i: (0, 0))
_b_spec = pl.BlockSpec((1, D), lambda i: (0, 0))
_col_spec = pl.BlockSpec((BM, 1), lambda i: (i, 0))
_s2_spec = pl.BlockSpec((1, 2), lambda i: (0, 0))


def _mlp_scale_body(s_ref, t_ref, ws1, bs1, ws2, bs2, wt1, bt1, wt2, bt2,
                    hr_ref, hc_ref,
                    s0_ref, t0_ref, s0s_ref, t0s_ref, oi_ref, ii_ref):
    x = s_ref[...]
    h = jnp.maximum(jnp.dot(x, ws1[...], preferred_element_type=jnp.float32)
                    + bs1[...], 0.0)
    s0 = jnp.dot(h, ws2[...], preferred_element_type=jnp.float32) + bs2[...]
    y = t_ref[...]
    g = jnp.maximum(jnp.dot(y, wt1[...], preferred_element_type=jnp.float32)
                    + bt1[...], 0.0)
    t0 = jnp.dot(g, wt2[...], preferred_element_type=jnp.float32) + bt2[...]
    oi = lax.rsqrt(hr_ref[...] + 1.0)
    ii = lax.rsqrt(hc_ref[...] + 1.0)
    s0_ref[...] = s0
    t0_ref[...] = t0
    s0s_ref[...] = s0 * oi
    t0s_ref[...] = t0 * ii
    oi_ref[...] = oi
    ii_ref[...] = ii


_mlp_scale = pl.pallas_call(
    _mlp_scale_body,
    grid=(GRID,),
    in_specs=[_row_spec, _row_spec,
              _w_spec, _b_spec, _w_spec, _b_spec,
              _w_spec, _b_spec, _w_spec, _b_spec,
              _col_spec, _col_spec],
    out_specs=[_row_spec, _row_spec, _row_spec, _row_spec,
               _col_spec, _col_spec],
    out_shape=[jax.ShapeDtypeStruct((N, D), jnp.float32)] * 4
    + [jax.ShapeDtypeStruct((N, 1), jnp.float32)] * 2,
)


def _combine_body(s_ref, t_ref, accs_ref, acct_ref, oi_ref, ii_ref, w_ref,
                  s1_ref, t1_ref, s1s_ref, t1s_ref):
    oi = oi_ref[...]
    ii = ii_ref[...]
    s1 = s_ref[...] + w_ref[0, 0] * (oi * accs_ref[...])
    t1 = t_ref[...] + w_ref[0, 1] * (ii * acct_ref[...])
    s1_ref[...] = s1
    t1_ref[...] = t1
    s1s_ref[...] = s1 * oi
    t1s_ref[...] = t1 * ii


_combine_scale = pl.pallas_call(
    _combine_body,
    grid=(GRID,),
    in_specs=[_row_spec, _row_spec, _row_spec, _row_spec,
              _col_spec, _col_spec, _s2_spec],
    out_specs=[_row_spec] * 4,
    out_shape=[jax.ShapeDtypeStruct((N, D), jnp.float32)] * 4,
)


def _final_body(s_ref, t_ref, accs_ref, acct_ref, oi_ref, ii_ref, w_ref,
                s2_ref, t2_ref):
    s2_ref[...] = s_ref[...] + w_ref[0, 0] * (oi_ref[...] * accs_ref[...])
    t2_ref[...] = t_ref[...] + w_ref[0, 1] * (ii_ref[...] * acct_ref[...])


_final_combine = pl.pallas_call(
    _final_body,
    grid=(GRID,),
    in_specs=[_row_spec, _row_spec, _row_spec, _row_spec,
              _col_spec, _col_spec, _s2_spec],
    out_specs=[_row_spec] * 2,
    out_shape=[jax.ShapeDtypeStruct((N, D), jnp.float32)] * 2,
)


def kernel(s, t, edge_index, Ws1, bs1, Ws2, bs2, Wt1, bt1, Wt2, bt2, w1, w2):
    row2d = edge_index[0].reshape(ROWS, CHUNK)
    col2d = edge_index[1].reshape(ROWS, CHUNK)

    hr, hc = _sc_degrees(row2d, col2d)
    hr = hr[:N].reshape(N, 1)
    hc = hc[:N].reshape(N, 1)

    s0, t0, s0s, t0s, oi, ii = _mlp_scale(
        s, t,
        Ws1.T, bs1.reshape(1, D), Ws2.T, bs2.reshape(1, D),
        Wt1.T, bt1.reshape(1, D), Wt2.T, bt2.reshape(1, D),
        hr, hc,
    )

    acc_s, acc_t = _sc_spmm(t0s, s0s, row2d, col2d)
    s1, t1, s1s, t1s = _combine_scale(s0, t0, acc_s, acc_t, oi, ii,
                                      w1.reshape(1, 2))
    acc_s2, acc_t2 = _sc_spmm(t1s, s1s, row2d, col2d)
    s2, t2 = _final_combine(s1, t1, acc_s2, acc_t2, oi, ii, w2.reshape(1, 2))
    return (s2, t2)
